# Initial kernel scaffold; baseline (speedup 1.0000x reference)
import jax
import jax.numpy as jnp
from jax.experimental import pallas as pl


def kernel(x, S, W1, b1, W2, b2):
    raise NotImplementedError("write your pallas kernel here")


# speedup vs baseline: 7.5288x; 7.5288x over previous
import functools

import jax
import jax.numpy as jnp
from jax.experimental import pallas as pl

_K = 16
_EPS = 0.1

_DIMNUM_T = (((0,), (0,)), ((), ()))
_DIMNUM_N = (((1,), (0,)), ((), ()))


def _topk_body(s_ref, pt_ref, degb_ref):
    s = s_ref[...]
    n = s.shape[0]
    rows = jax.lax.broadcasted_iota(jnp.int32, s.shape, 0)
    pt = jnp.zeros(s.shape, jnp.float32)
    for _ in range(_K):
        v = jnp.max(s, axis=0, keepdims=True)
        ismax = s == v
        amin = jnp.min(jnp.where(ismax, rows, n), axis=0, keepdims=True)
        sel = rows == amin
        pt = jnp.where(sel & (v > _EPS), 1.0, pt)
        s = jnp.where(sel, -jnp.inf, s)
    pt_ref[...] = pt
    degb_ref[...] = jnp.broadcast_to(
        jnp.sum(pt, axis=0, keepdims=True), degb_ref.shape)


def _rowsum_body(pt_ref, o_ref):
    c = pl.program_id(0)

    @pl.when(c == 0)
    def _():
        o_ref[...] = jnp.zeros_like(o_ref)

    part = jnp.sum(pt_ref[...], axis=1, keepdims=True)
    o_ref[...] = o_ref[...] + jnp.broadcast_to(part, o_ref.shape)


def _scaled_mm_body(pt_ref, v_ref, sc_ref, o_ref, *, ksteps, dimnum):
    k = pl.program_id(1)

    @pl.when(k == 0)
    def _():
        o_ref[...] = jnp.zeros_like(o_ref)

    o_ref[...] = o_ref[...] + jax.lax.dot_general(
        pt_ref[...], v_ref[...], dimnum,
        preferred_element_type=jnp.float32,
        precision=jax.lax.Precision.HIGHEST)

    @pl.when(k == ksteps - 1)
    def _():
        o_ref[...] = o_ref[...] * sc_ref[...][:, :1]


def _scaled_mm_tanh_body(pt_ref, v_ref, sc_ref, b_ref, o_ref, *, ksteps,
                         dimnum):
    k = pl.program_id(1)

    @pl.when(k == 0)
    def _():
        o_ref[...] = jnp.zeros_like(o_ref)

    o_ref[...] = o_ref[...] + jax.lax.dot_general(
        pt_ref[...], v_ref[...], dimnum,
        preferred_element_type=jnp.float32,
        precision=jax.lax.Precision.HIGHEST)

    @pl.when(k == ksteps - 1)
    def _():
        o_ref[...] = jnp.tanh(
            o_ref[...] * sc_ref[...][:, :1] + b_ref[0:1, :])


def _dense_relu_body(a_ref, w_ref, b_ref, o_ref):
    acc = jax.lax.dot_general(
        a_ref[...], w_ref[...], _DIMNUM_N,
        preferred_element_type=jnp.float32,
        precision=jax.lax.Precision.HIGHEST)
    o_ref[...] = jnp.maximum(acc + b_ref[0:1, :], 0.0)


def _mm_acc_body(a_ref, w_ref, o_ref, *, ksteps):
    k = pl.program_id(1)

    @pl.when(k == 0)
    def _():
        o_ref[...] = jnp.zeros_like(o_ref)

    o_ref[...] = o_ref[...] + jax.lax.dot_general(
        a_ref[...], w_ref[...], _DIMNUM_N,
        preferred_element_type=jnp.float32,
        precision=jax.lax.Precision.HIGHEST)


def kernel(x, S, W1, b1, W2, b2):
    n = S.shape[0]
    f = x.shape[1]
    hid = W1.shape[1]
    code = W2.shape[1]
    cpad = 128

    bc = 128
    pt, degb8 = pl.pallas_call(
        _topk_body,
        grid=(n // bc,),
        in_specs=[pl.BlockSpec((n, bc), lambda c: (0, c))],
        out_specs=[pl.BlockSpec((n, bc), lambda c: (0, c)),
                   pl.BlockSpec((8, bc), lambda c: (0, c))],
        out_shape=[jax.ShapeDtypeStruct((n, n), jnp.float32),
                   jax.ShapeDtypeStruct((8, n), jnp.float32)],
    )(S)

    bd = 256
    degd_w = pl.pallas_call(
        _rowsum_body,
        grid=(n // bd,),
        in_specs=[pl.BlockSpec((n, bd), lambda c: (0, c))],
        out_specs=pl.BlockSpec((n, 128), lambda c: (0, 0)),
        out_shape=jax.ShapeDtypeStruct((n, 128), jnp.float32),
    )(pt)

    degb = degb8[0]
    degd = degd_w[:, 0]
    binv = jnp.where(degb > 0, 1.0 / degb, 0.0)
    dinv = jnp.where(degd > 0, 1.0 / degd, 0.0)
    binv2 = jnp.broadcast_to(binv[:, None], (n, 128))
    dinv2 = jnp.broadcast_to(dinv[:, None], (n, 128))

    bi = 512
    bk = 512
    ks = n // bk

    he1 = pl.pallas_call(
        functools.partial(_scaled_mm_body, ksteps=ks, dimnum=_DIMNUM_T),
        grid=(n // bi, ks),
        in_specs=[pl.BlockSpec((bk, bi), lambda j, k: (k, j)),
                  pl.BlockSpec((bk, f), lambda j, k: (k, 0)),
                  pl.BlockSpec((bi, 128), lambda j, k: (j, 0))],
        out_specs=pl.BlockSpec((bi, f), lambda j, k: (j, 0)),
        out_shape=jax.ShapeDtypeStruct((n, f), jnp.float32),
    )(pt, x, binv2)
    agg1 = pl.pallas_call(
        functools.partial(_scaled_mm_body, ksteps=ks, dimnum=_DIMNUM_N),
        grid=(n // bi, ks),
        in_specs=[pl.BlockSpec((bi, bk), lambda i, k: (i, k)),
                  pl.BlockSpec((bk, f), lambda i, k: (k, 0)),
                  pl.BlockSpec((bi, 128), lambda i, k: (i, 0))],
        out_specs=pl.BlockSpec((bi, f), lambda i, k: (i, 0)),
        out_shape=jax.ShapeDtypeStruct((n, f), jnp.float32),
    )(pt, he1, dinv2)

    b1_2d = jnp.broadcast_to(b1[None, :], (8, hid))
    bj = 512
    feat = pl.pallas_call(
        _dense_relu_body,
        grid=(n // bi, hid // bj),
        in_specs=[pl.BlockSpec((bi, f), lambda i, j: (i, 0)),
                  pl.BlockSpec((f, bj), lambda i, j: (0, j)),
                  pl.BlockSpec((8, bj), lambda i, j: (0, j))],
        out_specs=pl.BlockSpec((bi, bj), lambda i, j: (i, j)),
        out_shape=jax.ShapeDtypeStruct((n, hid), jnp.float32),
    )(agg1, W1, b1_2d)

    w2p = jnp.pad(W2, ((0, 0), (0, cpad - code)))
    b2p = jnp.broadcast_to(jnp.pad(b2, (0, cpad - code))[None, :], (8, cpad))
    xt2 = pl.pallas_call(
        functools.partial(_mm_acc_body, ksteps=hid // bk),
        grid=(n // bi, hid // bk),
        in_specs=[pl.BlockSpec((bi, bk), lambda i, k: (i, k)),
                  pl.BlockSpec((bk, cpad), lambda i, k: (k, 0))],
        out_specs=pl.BlockSpec((bi, cpad), lambda i, k: (i, 0)),
        out_shape=jax.ShapeDtypeStruct((n, cpad), jnp.float32),
    )(feat, w2p)
    he2 = pl.pallas_call(
        functools.partial(_scaled_mm_body, ksteps=ks, dimnum=_DIMNUM_T),
        grid=(n // bi, ks),
        in_specs=[pl.BlockSpec((bk, bi), lambda j, k: (k, j)),
                  pl.BlockSpec((bk, cpad), lambda j, k: (k, 0)),
                  pl.BlockSpec((bi, 128), lambda j, k: (j, 0))],
        out_specs=pl.BlockSpec((bi, cpad), lambda j, k: (j, 0)),
        out_shape=jax.ShapeDtypeStruct((n, cpad), jnp.float32),
    )(pt, xt2, binv2)
    code_pad = pl.pallas_call(
        functools.partial(_scaled_mm_tanh_body, ksteps=ks, dimnum=_DIMNUM_N),
        grid=(n // bi, ks),
        in_specs=[pl.BlockSpec((bi, bk), lambda i, k: (i, k)),
                  pl.BlockSpec((bk, cpad), lambda i, k: (k, 0)),
                  pl.BlockSpec((bi, 128), lambda i, k: (i, 0)),
                  pl.BlockSpec((8, cpad), lambda i, k: (0, 0))],
        out_specs=pl.BlockSpec((bi, cpad), lambda i, k: (i, 0)),
        out_shape=jax.ShapeDtypeStruct((n, cpad), jnp.float32),
    )(pt, he2, dinv2, b2p)

    return code_pad[:, :code]


# speedup vs baseline: 10.3688x; 1.3772x over previous
import functools

import jax
import jax.numpy as jnp
from jax.experimental import pallas as pl

_K = 16
_EPS = 0.1

_DIMNUM_T = (((0,), (0,)), ((), ()))
_DIMNUM_N = (((1,), (0,)), ((), ()))


def _topk_body(s_ref, pt_ref, degb_ref):
    s0 = s_ref[...]
    n = s0.shape[0]
    rows = jax.lax.broadcasted_iota(jnp.int32, s0.shape, 0)
    s = s0
    for _ in range(_K):
        v = jnp.max(s, axis=0, keepdims=True)
        amin = jnp.min(jnp.where(s == v, rows, n), axis=0, keepdims=True)
        s = jnp.where(rows == amin, -jnp.inf, s)
    pt = jnp.where((s != s0) & (s0 > _EPS), 1.0, 0.0)
    pt_ref[...] = pt
    degb_ref[...] = jnp.broadcast_to(
        jnp.sum(pt, axis=0, keepdims=True), degb_ref.shape)


def _rowsum_body(pt_ref, o_ref):
    c = pl.program_id(0)

    @pl.when(c == 0)
    def _():
        o_ref[...] = jnp.zeros_like(o_ref)

    part = jnp.sum(pt_ref[...], axis=1, keepdims=True)
    o_ref[...] = o_ref[...] + jnp.broadcast_to(part, o_ref.shape)


def _scaled_mm_body(pt_ref, v_ref, sc_ref, o_ref, *, ksteps, dimnum):
    k = pl.program_id(1)

    @pl.when(k == 0)
    def _():
        o_ref[...] = jnp.zeros_like(o_ref)

    o_ref[...] = o_ref[...] + jax.lax.dot_general(
        pt_ref[...], v_ref[...], dimnum,
        preferred_element_type=jnp.float32,
        precision=jax.lax.Precision.DEFAULT)

    @pl.when(k == ksteps - 1)
    def _():
        o_ref[...] = o_ref[...] * sc_ref[...][:, :1]


def _scaled_mm_tanh_body(pt_ref, v_ref, sc_ref, b_ref, o_ref, *, ksteps,
                         dimnum):
    k = pl.program_id(1)

    @pl.when(k == 0)
    def _():
        o_ref[...] = jnp.zeros_like(o_ref)

    o_ref[...] = o_ref[...] + jax.lax.dot_general(
        pt_ref[...], v_ref[...], dimnum,
        preferred_element_type=jnp.float32,
        precision=jax.lax.Precision.DEFAULT)

    @pl.when(k == ksteps - 1)
    def _():
        o_ref[...] = jnp.tanh(
            o_ref[...] * sc_ref[...][:, :1] + b_ref[0:1, :])


def _dense_relu_body(a_ref, w_ref, b_ref, o_ref):
    acc = jax.lax.dot_general(
        a_ref[...], w_ref[...], _DIMNUM_N,
        preferred_element_type=jnp.float32,
        precision=jax.lax.Precision.DEFAULT)
    o_ref[...] = jnp.maximum(acc + b_ref[0:1, :], 0.0)


def _mm_acc_body(a_ref, w_ref, o_ref, *, ksteps):
    k = pl.program_id(1)

    @pl.when(k == 0)
    def _():
        o_ref[...] = jnp.zeros_like(o_ref)

    o_ref[...] = o_ref[...] + jax.lax.dot_general(
        a_ref[...], w_ref[...], _DIMNUM_N,
        preferred_element_type=jnp.float32,
        precision=jax.lax.Precision.DEFAULT)


def kernel(x, S, W1, b1, W2, b2):
    n = S.shape[0]
    f = x.shape[1]
    hid = W1.shape[1]
    code = W2.shape[1]
    cpad = 128

    bc = 128
    pt, degb8 = pl.pallas_call(
        _topk_body,
        grid=(n // bc,),
        in_specs=[pl.BlockSpec((n, bc), lambda c: (0, c))],
        out_specs=[pl.BlockSpec((n, bc), lambda c: (0, c)),
                   pl.BlockSpec((8, bc), lambda c: (0, c))],
        out_shape=[jax.ShapeDtypeStruct((n, n), jnp.float32),
                   jax.ShapeDtypeStruct((8, n), jnp.float32)],
    )(S)

    bd = 256
    degd_w = pl.pallas_call(
        _rowsum_body,
        grid=(n // bd,),
        in_specs=[pl.BlockSpec((n, bd), lambda c: (0, c))],
        out_specs=pl.BlockSpec((n, 128), lambda c: (0, 0)),
        out_shape=jax.ShapeDtypeStruct((n, 128), jnp.float32),
    )(pt)

    degb = degb8[0]
    degd = degd_w[:, 0]
    binv = jnp.where(degb > 0, 1.0 / degb, 0.0)
    dinv = jnp.where(degd > 0, 1.0 / degd, 0.0)
    binv2 = jnp.broadcast_to(binv[:, None], (n, 128))
    dinv2 = jnp.broadcast_to(dinv[:, None], (n, 128))

    bi = 512
    bk = 512
    ks = n // bk

    he1 = pl.pallas_call(
        functools.partial(_scaled_mm_body, ksteps=ks, dimnum=_DIMNUM_T),
        grid=(n // bi, ks),
        in_specs=[pl.BlockSpec((bk, bi), lambda j, k: (k, j)),
                  pl.BlockSpec((bk, f), lambda j, k: (k, 0)),
                  pl.BlockSpec((bi, 128), lambda j, k: (j, 0))],
        out_specs=pl.BlockSpec((bi, f), lambda j, k: (j, 0)),
        out_shape=jax.ShapeDtypeStruct((n, f), jnp.float32),
    )(pt, x, binv2)
    agg1 = pl.pallas_call(
        functools.partial(_scaled_mm_body, ksteps=ks, dimnum=_DIMNUM_N),
        grid=(n // bi, ks),
        in_specs=[pl.BlockSpec((bi, bk), lambda i, k: (i, k)),
                  pl.BlockSpec((bk, f), lambda i, k: (k, 0)),
                  pl.BlockSpec((bi, 128), lambda i, k: (i, 0))],
        out_specs=pl.BlockSpec((bi, f), lambda i, k: (i, 0)),
        out_shape=jax.ShapeDtypeStruct((n, f), jnp.float32),
    )(pt, he1, dinv2)

    b1_2d = jnp.broadcast_to(b1[None, :], (8, hid))
    bj = 512
    feat = pl.pallas_call(
        _dense_relu_body,
        grid=(n // bi, hid // bj),
        in_specs=[pl.BlockSpec((bi, f), lambda i, j: (i, 0)),
                  pl.BlockSpec((f, bj), lambda i, j: (0, j)),
                  pl.BlockSpec((8, bj), lambda i, j: (0, j))],
        out_specs=pl.BlockSpec((bi, bj), lambda i, j: (i, j)),
        out_shape=jax.ShapeDtypeStruct((n, hid), jnp.float32),
    )(agg1, W1, b1_2d)

    w2p = jnp.pad(W2, ((0, 0), (0, cpad - code)))
    b2p = jnp.broadcast_to(jnp.pad(b2, (0, cpad - code))[None, :], (8, cpad))
    xt2 = pl.pallas_call(
        functools.partial(_mm_acc_body, ksteps=hid // bk),
        grid=(n // bi, hid // bk),
        in_specs=[pl.BlockSpec((bi, bk), lambda i, k: (i, k)),
                  pl.BlockSpec((bk, cpad), lambda i, k: (k, 0))],
        out_specs=pl.BlockSpec((bi, cpad), lambda i, k: (i, 0)),
        out_shape=jax.ShapeDtypeStruct((n, cpad), jnp.float32),
    )(feat, w2p)
    he2 = pl.pallas_call(
        functools.partial(_scaled_mm_body, ksteps=ks, dimnum=_DIMNUM_T),
        grid=(n // bi, ks),
        in_specs=[pl.BlockSpec((bk, bi), lambda j, k: (k, j)),
                  pl.BlockSpec((bk, cpad), lambda j, k: (k, 0)),
                  pl.BlockSpec((bi, 128), lambda j, k: (j, 0))],
        out_specs=pl.BlockSpec((bi, cpad), lambda j, k: (j, 0)),
        out_shape=jax.ShapeDtypeStruct((n, cpad), jnp.float32),
    )(pt, xt2, binv2)
    code_pad = pl.pallas_call(
        functools.partial(_scaled_mm_tanh_body, ksteps=ks, dimnum=_DIMNUM_N),
        grid=(n // bi, ks),
        in_specs=[pl.BlockSpec((bi, bk), lambda i, k: (i, k)),
                  pl.BlockSpec((bk, cpad), lambda i, k: (k, 0)),
                  pl.BlockSpec((bi, 128), lambda i, k: (i, 0)),
                  pl.BlockSpec((8, cpad), lambda i, k: (0, 0))],
        out_specs=pl.BlockSpec((bi, cpad), lambda i, k: (i, 0)),
        out_shape=jax.ShapeDtypeStruct((n, cpad), jnp.float32),
    )(pt, he2, dinv2, b2p)

    return code_pad[:, :code]


# speedup vs baseline: 10.8090x; 1.0424x over previous
import functools

import jax
import jax.numpy as jnp
from jax.experimental import pallas as pl
from jax.experimental.pallas import tpu as pltpu

_K = 16
_EPS = 0.1

_DIMNUM_T = (((0,), (0,)), ((), ()))
_DIMNUM_N = (((1,), (0,)), ((), ()))


def _tree_max(a):
    r = a.shape[0]
    while r > 8:
        h = r // 2
        a = jnp.maximum(a[:h], a[h:])
        r = h
    return jnp.max(a, axis=0, keepdims=True)


def _tree_min(a):
    r = a.shape[0]
    while r > 8:
        h = r // 2
        a = jnp.minimum(a[:h], a[h:])
        r = h
    return jnp.min(a, axis=0, keepdims=True)


def _topk_body(s_ref, pt_ref, degb_ref):
    s0 = s_ref[...]
    n = s0.shape[0]
    rows = jax.lax.broadcasted_iota(jnp.int32, s0.shape, 0)
    s = s0
    for _ in range(_K):
        v = _tree_max(s)
        amin = _tree_min(jnp.where(s == v, rows, n))
        s = jnp.where(rows == amin, -jnp.inf, s)
    pt = jnp.where((s != s0) & (s0 > _EPS), 1.0, 0.0)
    pt_ref[...] = pt.astype(pt_ref.dtype)
    degb_ref[...] = jnp.broadcast_to(
        jnp.sum(pt, axis=0, keepdims=True), degb_ref.shape)


def _rowsum_body(pt_ref, o_ref):
    c = pl.program_id(0)

    @pl.when(c == 0)
    def _():
        o_ref[...] = jnp.zeros_like(o_ref)

    part = jnp.sum(pt_ref[...].astype(jnp.float32), axis=1, keepdims=True)
    o_ref[...] = o_ref[...] + jnp.broadcast_to(part, o_ref.shape)


def _scaled_mm_body(pt_ref, v_ref, sc_ref, o_ref, acc_ref, *, ksteps, dimnum):
    k = pl.program_id(1)

    @pl.when(k == 0)
    def _():
        acc_ref[...] = jnp.zeros_like(acc_ref)

    acc_ref[...] = acc_ref[...] + jax.lax.dot_general(
        pt_ref[...], v_ref[...], dimnum,
        preferred_element_type=jnp.float32)

    @pl.when(k == ksteps - 1)
    def _():
        o_ref[...] = (acc_ref[...] * sc_ref[...][:, :1]).astype(o_ref.dtype)


def _scaled_mm_tanh_body(pt_ref, v_ref, sc_ref, b_ref, o_ref, acc_ref, *,
                         ksteps, dimnum):
    k = pl.program_id(1)

    @pl.when(k == 0)
    def _():
        acc_ref[...] = jnp.zeros_like(acc_ref)

    acc_ref[...] = acc_ref[...] + jax.lax.dot_general(
        pt_ref[...], v_ref[...], dimnum,
        preferred_element_type=jnp.float32)

    @pl.when(k == ksteps - 1)
    def _():
        o_ref[...] = jnp.tanh(
            acc_ref[...] * sc_ref[...][:, :1] + b_ref[0:1, :])


def _dense_relu_body(a_ref, w_ref, b_ref, o_ref):
    acc = jax.lax.dot_general(
        a_ref[...], w_ref[...], _DIMNUM_N,
        preferred_element_type=jnp.float32)
    o_ref[...] = jnp.maximum(acc + b_ref[0:1, :], 0.0).astype(o_ref.dtype)


def _mm_acc_body(a_ref, w_ref, o_ref, acc_ref, *, ksteps):
    k = pl.program_id(1)

    @pl.when(k == 0)
    def _():
        acc_ref[...] = jnp.zeros_like(acc_ref)

    acc_ref[...] = acc_ref[...] + jax.lax.dot_general(
        a_ref[...], w_ref[...], _DIMNUM_N,
        preferred_element_type=jnp.float32)

    @pl.when(k == ksteps - 1)
    def _():
        o_ref[...] = acc_ref[...].astype(o_ref.dtype)


def kernel(x, S, W1, b1, W2, b2):
    n = S.shape[0]
    f = x.shape[1]
    hid = W1.shape[1]
    code = W2.shape[1]
    cpad = 128

    bc = 128
    pt, degb8 = pl.pallas_call(
        _topk_body,
        grid=(n // bc,),
        in_specs=[pl.BlockSpec((n, bc), lambda c: (0, c))],
        out_specs=[pl.BlockSpec((n, bc), lambda c: (0, c)),
                   pl.BlockSpec((8, bc), lambda c: (0, c))],
        out_shape=[jax.ShapeDtypeStruct((n, n), jnp.bfloat16),
                   jax.ShapeDtypeStruct((8, n), jnp.float32)],
    )(S)

    bd = 256
    degd_w = pl.pallas_call(
        _rowsum_body,
        grid=(n // bd,),
        in_specs=[pl.BlockSpec((n, bd), lambda c: (0, c))],
        out_specs=pl.BlockSpec((n, 128), lambda c: (0, 0)),
        out_shape=jax.ShapeDtypeStruct((n, 128), jnp.float32),
    )(pt)

    degb = degb8[0]
    degd = degd_w[:, 0]
    binv = jnp.where(degb > 0, 1.0 / degb, 0.0)
    dinv = jnp.where(degd > 0, 1.0 / degd, 0.0)
    binv2 = jnp.broadcast_to(binv[:, None], (n, 128))
    dinv2 = jnp.broadcast_to(dinv[:, None], (n, 128))

    bi = 512
    bk = 512
    ks = n // bk
    x_b = x.astype(jnp.bfloat16)
    w1_b = W1.astype(jnp.bfloat16)

    he1 = pl.pallas_call(
        functools.partial(_scaled_mm_body, ksteps=ks, dimnum=_DIMNUM_T),
        grid=(n // bi, ks),
        in_specs=[pl.BlockSpec((bk, bi), lambda j, k: (k, j)),
                  pl.BlockSpec((bk, f), lambda j, k: (k, 0)),
                  pl.BlockSpec((bi, 128), lambda j, k: (j, 0))],
        out_specs=pl.BlockSpec((bi, f), lambda j, k: (j, 0)),
        out_shape=jax.ShapeDtypeStruct((n, f), jnp.bfloat16),
        scratch_shapes=[pltpu.VMEM((bi, f), jnp.float32)],
    )(pt, x_b, binv2)
    agg1 = pl.pallas_call(
        functools.partial(_scaled_mm_body, ksteps=ks, dimnum=_DIMNUM_N),
        grid=(n // bi, ks),
        in_specs=[pl.BlockSpec((bi, bk), lambda i, k: (i, k)),
                  pl.BlockSpec((bk, f), lambda i, k: (k, 0)),
                  pl.BlockSpec((bi, 128), lambda i, k: (i, 0))],
        out_specs=pl.BlockSpec((bi, f), lambda i, k: (i, 0)),
        out_shape=jax.ShapeDtypeStruct((n, f), jnp.bfloat16),
        scratch_shapes=[pltpu.VMEM((bi, f), jnp.float32)],
    )(pt, he1, dinv2)

    b1_2d = jnp.broadcast_to(b1[None, :], (8, hid))
    bj = 512
    feat = pl.pallas_call(
        _dense_relu_body,
        grid=(n // bi, hid // bj),
        in_specs=[pl.BlockSpec((bi, f), lambda i, j: (i, 0)),
                  pl.BlockSpec((f, bj), lambda i, j: (0, j)),
                  pl.BlockSpec((8, bj), lambda i, j: (0, j))],
        out_specs=pl.BlockSpec((bi, bj), lambda i, j: (i, j)),
        out_shape=jax.ShapeDtypeStruct((n, hid), jnp.bfloat16),
    )(agg1, w1_b, b1_2d)

    w2p = jnp.pad(W2, ((0, 0), (0, cpad - code))).astype(jnp.bfloat16)
    b2p = jnp.broadcast_to(jnp.pad(b2, (0, cpad - code))[None, :], (8, cpad))
    xt2 = pl.pallas_call(
        functools.partial(_mm_acc_body, ksteps=hid // bk),
        grid=(n // bi, hid // bk),
        in_specs=[pl.BlockSpec((bi, bk), lambda i, k: (i, k)),
                  pl.BlockSpec((bk, cpad), lambda i, k: (k, 0))],
        out_specs=pl.BlockSpec((bi, cpad), lambda i, k: (i, 0)),
        out_shape=jax.ShapeDtypeStruct((n, cpad), jnp.bfloat16),
        scratch_shapes=[pltpu.VMEM((bi, cpad), jnp.float32)],
    )(feat, w2p)
    he2 = pl.pallas_call(
        functools.partial(_scaled_mm_body, ksteps=ks, dimnum=_DIMNUM_T),
        grid=(n // bi, ks),
        in_specs=[pl.BlockSpec((bk, bi), lambda j, k: (k, j)),
                  pl.BlockSpec((bk, cpad), lambda j, k: (k, 0)),
                  pl.BlockSpec((bi, 128), lambda j, k: (j, 0))],
        out_specs=pl.BlockSpec((bi, cpad), lambda j, k: (j, 0)),
        out_shape=jax.ShapeDtypeStruct((n, cpad), jnp.bfloat16),
        scratch_shapes=[pltpu.VMEM((bi, cpad), jnp.float32)],
    )(pt, xt2, binv2)
    code_pad = pl.pallas_call(
        functools.partial(_scaled_mm_tanh_body, ksteps=ks, dimnum=_DIMNUM_N),
        grid=(n // bi, ks),
        in_specs=[pl.BlockSpec((bi, bk), lambda i, k: (i, k)),
                  pl.BlockSpec((bk, cpad), lambda i, k: (k, 0)),
                  pl.BlockSpec((bi, 128), lambda i, k: (i, 0)),
                  pl.BlockSpec((8, cpad), lambda i, k: (0, 0))],
        out_specs=pl.BlockSpec((bi, cpad), lambda i, k: (i, 0)),
        out_shape=jax.ShapeDtypeStruct((n, cpad), jnp.float32),
        scratch_shapes=[pltpu.VMEM((bi, cpad), jnp.float32)],
    )(pt, he2, dinv2, b2p)

    return code_pad[:, :code]


# speedup vs baseline: 10.9893x; 1.0167x over previous
import functools

import jax
import jax.numpy as jnp
from jax.experimental import pallas as pl
from jax.experimental.pallas import tpu as pltpu

_K = 16
_EPS = 0.1

_DIMNUM_T = (((0,), (0,)), ((), ()))
_DIMNUM_N = (((1,), (0,)), ((), ()))


def _tree_min(a):
    r = a.shape[0]
    while r > 8:
        h = r // 2
        a = jnp.minimum(a[:h], a[h:])
        r = h
    return jnp.min(a, axis=0, keepdims=True)


def _merge2max(a1, a2, b1, b2):
    return (jnp.maximum(a1, b1),
            jnp.maximum(jnp.minimum(a1, b1), jnp.maximum(a2, b2)))


def _merge2min(a1, a2, b1, b2):
    return (jnp.minimum(a1, b1),
            jnp.minimum(jnp.maximum(a1, b1), jnp.minimum(a2, b2)))


def _vtop2(s):
    h = s.shape[0] // 8
    p = [s[i * h:(i + 1) * h] for i in range(8)]
    hi = [jnp.maximum(p[i], p[i + 1]) for i in (0, 2, 4, 6)]
    lo = [jnp.minimum(p[i], p[i + 1]) for i in (0, 2, 4, 6)]
    x1, x2 = _merge2max(hi[0], lo[0], hi[1], lo[1])
    y1, y2 = _merge2max(hi[2], lo[2], hi[3], lo[3])
    v1, v2 = _merge2max(x1, x2, y1, y2)
    while v1.shape[0] > 1:
        m = v1.shape[0] // 2
        v1, v2 = _merge2max(v1[:m], v2[:m], v1[m:], v2[m:])
    return v1, v2


def _rbot2(a):
    h = a.shape[0] // 8
    p = [a[i * h:(i + 1) * h] for i in range(8)]
    lo = [jnp.minimum(p[i], p[i + 1]) for i in (0, 2, 4, 6)]
    hi = [jnp.maximum(p[i], p[i + 1]) for i in (0, 2, 4, 6)]
    x1, x2 = _merge2min(lo[0], hi[0], lo[1], hi[1])
    y1, y2 = _merge2min(lo[2], hi[2], lo[3], hi[3])
    r1, r2 = _merge2min(x1, x2, y1, y2)
    while r1.shape[0] > 1:
        m = r1.shape[0] // 2
        r1, r2 = _merge2min(r1[:m], r2[:m], r1[m:], r2[m:])
    return r1, r2


def _topk_body(s_ref, pt_ref, degb_ref):
    s0 = s_ref[...]
    n = s0.shape[0]
    rows = jax.lax.broadcasted_iota(jnp.int32, s0.shape, 0)
    s = s0
    for _ in range(_K // 2):
        v1, v2 = _vtop2(s)
        a1 = _tree_min(jnp.where(s == v1, rows, n))
        k1, k2 = _rbot2(jnp.where(s == v2, rows, n))
        a2 = jnp.where(v1 == v2, k2, k1)
        s = jnp.where((rows == a1) | (rows == a2), -jnp.inf, s)
    pt = jnp.where((s != s0) & (s0 > _EPS), 1.0, 0.0)
    pt_ref[...] = pt.astype(pt_ref.dtype)
    degb_ref[...] = jnp.broadcast_to(
        jnp.sum(pt, axis=0, keepdims=True), degb_ref.shape)


def _rowsum_body(pt_ref, o_ref):
    c = pl.program_id(0)

    @pl.when(c == 0)
    def _():
        o_ref[...] = jnp.zeros_like(o_ref)

    part = jnp.sum(pt_ref[...].astype(jnp.float32), axis=1, keepdims=True)
    o_ref[...] = o_ref[...] + jnp.broadcast_to(part, o_ref.shape)


def _scaled_mm_body(pt_ref, v_ref, sc_ref, o_ref, acc_ref, *, ksteps, dimnum):
    k = pl.program_id(1)

    @pl.when(k == 0)
    def _():
        acc_ref[...] = jnp.zeros_like(acc_ref)

    acc_ref[...] = acc_ref[...] + jax.lax.dot_general(
        pt_ref[...], v_ref[...], dimnum,
        preferred_element_type=jnp.float32)

    @pl.when(k == ksteps - 1)
    def _():
        o_ref[...] = (acc_ref[...] * sc_ref[...][:, :1]).astype(o_ref.dtype)


def _scaled_mm_tanh_body(pt_ref, v_ref, sc_ref, b_ref, o_ref, acc_ref, *,
                         ksteps, dimnum):
    k = pl.program_id(1)

    @pl.when(k == 0)
    def _():
        acc_ref[...] = jnp.zeros_like(acc_ref)

    acc_ref[...] = acc_ref[...] + jax.lax.dot_general(
        pt_ref[...], v_ref[...], dimnum,
        preferred_element_type=jnp.float32)

    @pl.when(k == ksteps - 1)
    def _():
        o_ref[...] = jnp.tanh(
            acc_ref[...] * sc_ref[...][:, :1] + b_ref[0:1, :])


def _dense_relu_body(a_ref, w_ref, b_ref, o_ref):
    acc = jax.lax.dot_general(
        a_ref[...], w_ref[...], _DIMNUM_N,
        preferred_element_type=jnp.float32)
    o_ref[...] = jnp.maximum(acc + b_ref[0:1, :], 0.0).astype(o_ref.dtype)


def _mm_acc_body(a_ref, w_ref, o_ref, acc_ref, *, ksteps):
    k = pl.program_id(1)

    @pl.when(k == 0)
    def _():
        acc_ref[...] = jnp.zeros_like(acc_ref)

    acc_ref[...] = acc_ref[...] + jax.lax.dot_general(
        a_ref[...], w_ref[...], _DIMNUM_N,
        preferred_element_type=jnp.float32)

    @pl.when(k == ksteps - 1)
    def _():
        o_ref[...] = acc_ref[...].astype(o_ref.dtype)


def kernel(x, S, W1, b1, W2, b2):
    n = S.shape[0]
    f = x.shape[1]
    hid = W1.shape[1]
    code = W2.shape[1]
    cpad = 128

    bc = 128
    pt, degb8 = pl.pallas_call(
        _topk_body,
        grid=(n // bc,),
        in_specs=[pl.BlockSpec((n, bc), lambda c: (0, c))],
        out_specs=[pl.BlockSpec((n, bc), lambda c: (0, c)),
                   pl.BlockSpec((8, bc), lambda c: (0, c))],
        out_shape=[jax.ShapeDtypeStruct((n, n), jnp.bfloat16),
                   jax.ShapeDtypeStruct((8, n), jnp.float32)],
    )(S)

    bd = 256
    degd_w = pl.pallas_call(
        _rowsum_body,
        grid=(n // bd,),
        in_specs=[pl.BlockSpec((n, bd), lambda c: (0, c))],
        out_specs=pl.BlockSpec((n, 128), lambda c: (0, 0)),
        out_shape=jax.ShapeDtypeStruct((n, 128), jnp.float32),
    )(pt)

    degb = degb8[0]
    degd = degd_w[:, 0]
    binv = jnp.where(degb > 0, 1.0 / degb, 0.0)
    dinv = jnp.where(degd > 0, 1.0 / degd, 0.0)
    binv2 = jnp.broadcast_to(binv[:, None], (n, 128))
    dinv2 = jnp.broadcast_to(dinv[:, None], (n, 128))

    bi = 512
    bk = 512
    ks = n // bk
    x_b = x.astype(jnp.bfloat16)
    w1_b = W1.astype(jnp.bfloat16)

    he1 = pl.pallas_call(
        functools.partial(_scaled_mm_body, ksteps=ks, dimnum=_DIMNUM_T),
        grid=(n // bi, ks),
        in_specs=[pl.BlockSpec((bk, bi), lambda j, k: (k, j)),
                  pl.BlockSpec((bk, f), lambda j, k: (k, 0)),
                  pl.BlockSpec((bi, 128), lambda j, k: (j, 0))],
        out_specs=pl.BlockSpec((bi, f), lambda j, k: (j, 0)),
        out_shape=jax.ShapeDtypeStruct((n, f), jnp.bfloat16),
        scratch_shapes=[pltpu.VMEM((bi, f), jnp.float32)],
    )(pt, x_b, binv2)
    agg1 = pl.pallas_call(
        functools.partial(_scaled_mm_body, ksteps=ks, dimnum=_DIMNUM_N),
        grid=(n // bi, ks),
        in_specs=[pl.BlockSpec((bi, bk), lambda i, k: (i, k)),
                  pl.BlockSpec((bk, f), lambda i, k: (k, 0)),
                  pl.BlockSpec((bi, 128), lambda i, k: (i, 0))],
        out_specs=pl.BlockSpec((bi, f), lambda i, k: (i, 0)),
        out_shape=jax.ShapeDtypeStruct((n, f), jnp.bfloat16),
        scratch_shapes=[pltpu.VMEM((bi, f), jnp.float32)],
    )(pt, he1, dinv2)

    b1_2d = jnp.broadcast_to(b1[None, :], (8, hid))
    bj = 512
    feat = pl.pallas_call(
        _dense_relu_body,
        grid=(n // bi, hid // bj),
        in_specs=[pl.BlockSpec((bi, f), lambda i, j: (i, 0)),
                  pl.BlockSpec((f, bj), lambda i, j: (0, j)),
                  pl.BlockSpec((8, bj), lambda i, j: (0, j))],
        out_specs=pl.BlockSpec((bi, bj), lambda i, j: (i, j)),
        out_shape=jax.ShapeDtypeStruct((n, hid), jnp.bfloat16),
    )(agg1, w1_b, b1_2d)

    w2p = jnp.pad(W2, ((0, 0), (0, cpad - code))).astype(jnp.bfloat16)
    b2p = jnp.broadcast_to(jnp.pad(b2, (0, cpad - code))[None, :], (8, cpad))
    xt2 = pl.pallas_call(
        functools.partial(_mm_acc_body, ksteps=hid // bk),
        grid=(n // bi, hid // bk),
        in_specs=[pl.BlockSpec((bi, bk), lambda i, k: (i, k)),
                  pl.BlockSpec((bk, cpad), lambda i, k: (k, 0))],
        out_specs=pl.BlockSpec((bi, cpad), lambda i, k: (i, 0)),
        out_shape=jax.ShapeDtypeStruct((n, cpad), jnp.bfloat16),
        scratch_shapes=[pltpu.VMEM((bi, cpad), jnp.float32)],
    )(feat, w2p)
    he2 = pl.pallas_call(
        functools.partial(_scaled_mm_body, ksteps=ks, dimnum=_DIMNUM_T),
        grid=(n // bi, ks),
        in_specs=[pl.BlockSpec((bk, bi), lambda j, k: (k, j)),
                  pl.BlockSpec((bk, cpad), lambda j, k: (k, 0)),
                  pl.BlockSpec((bi, 128), lambda j, k: (j, 0))],
        out_specs=pl.BlockSpec((bi, cpad), lambda j, k: (j, 0)),
        out_shape=jax.ShapeDtypeStruct((n, cpad), jnp.bfloat16),
        scratch_shapes=[pltpu.VMEM((bi, cpad), jnp.float32)],
    )(pt, xt2, binv2)
    code_pad = pl.pallas_call(
        functools.partial(_scaled_mm_tanh_body, ksteps=ks, dimnum=_DIMNUM_N),
        grid=(n // bi, ks),
        in_specs=[pl.BlockSpec((bi, bk), lambda i, k: (i, k)),
                  pl.BlockSpec((bk, cpad), lambda i, k: (k, 0)),
                  pl.BlockSpec((bi, 128), lambda i, k: (i, 0)),
                  pl.BlockSpec((8, cpad), lambda i, k: (0, 0))],
        out_specs=pl.BlockSpec((bi, cpad), lambda i, k: (i, 0)),
        out_shape=jax.ShapeDtypeStruct((n, cpad), jnp.float32),
        scratch_shapes=[pltpu.VMEM((bi, cpad), jnp.float32)],
    )(pt, he2, dinv2, b2p)

    return code_pad[:, :code]


# speedup vs baseline: 13.7394x; 1.2503x over previous
import functools

import jax
import jax.numpy as jnp
from jax.experimental import pallas as pl
from jax.experimental.pallas import tpu as pltpu

_K = 16
_EPS = 0.1

_DIMNUM_T = (((0,), (0,)), ((), ()))
_DIMNUM_N = (((1,), (0,)), ((), ()))


def _tree_min(a):
    r = a.shape[0]
    while r > 8:
        h = r // 2
        a = jnp.minimum(a[:h], a[h:])
        r = h
    return jnp.min(a, axis=0, keepdims=True)


def _merge2max(a1, a2, b1, b2):
    return (jnp.maximum(a1, b1),
            jnp.maximum(jnp.minimum(a1, b1), jnp.maximum(a2, b2)))


def _merge2min(a1, a2, b1, b2):
    return (jnp.minimum(a1, b1),
            jnp.minimum(jnp.maximum(a1, b1), jnp.minimum(a2, b2)))


def _vtop2(s):
    h = s.shape[0] // 8
    p = [s[i * h:(i + 1) * h] for i in range(8)]
    hi = [jnp.maximum(p[i], p[i + 1]) for i in (0, 2, 4, 6)]
    lo = [jnp.minimum(p[i], p[i + 1]) for i in (0, 2, 4, 6)]
    x1, x2 = _merge2max(hi[0], lo[0], hi[1], lo[1])
    y1, y2 = _merge2max(hi[2], lo[2], hi[3], lo[3])
    v1, v2 = _merge2max(x1, x2, y1, y2)
    while v1.shape[0] > 1:
        m = v1.shape[0] // 2
        v1, v2 = _merge2max(v1[:m], v2[:m], v1[m:], v2[m:])
    return v1, v2


def _rbot2(a):
    h = a.shape[0] // 8
    p = [a[i * h:(i + 1) * h] for i in range(8)]
    lo = [jnp.minimum(p[i], p[i + 1]) for i in (0, 2, 4, 6)]
    hi = [jnp.maximum(p[i], p[i + 1]) for i in (0, 2, 4, 6)]
    x1, x2 = _merge2min(lo[0], hi[0], lo[1], hi[1])
    y1, y2 = _merge2min(lo[2], hi[2], lo[3], hi[3])
    r1, r2 = _merge2min(x1, x2, y1, y2)
    while r1.shape[0] > 1:
        m = r1.shape[0] // 2
        r1, r2 = _merge2min(r1[:m], r2[:m], r1[m:], r2[m:])
    return r1, r2


def _topk_body(s_ref, pt_ref, degb_ref):
    s0 = s_ref[...]
    n = s0.shape[0]
    rows = jax.lax.broadcasted_iota(jnp.int32, s0.shape, 0)
    s = s0
    for _ in range(_K // 2):
        v1, v2 = _vtop2(s)
        a1 = _tree_min(jnp.where(s == v1, rows, n))
        k1, k2 = _rbot2(jnp.where(s == v2, rows, n))
        a2 = jnp.where(v1 == v2, k2, k1)
        s = jnp.where((rows == a1) | (rows == a2), -jnp.inf, s)
    pt = jnp.where((s != s0) & (s0 > _EPS), 1.0, 0.0)
    pt_ref[...] = pt.astype(pt_ref.dtype)
    degb_ref[...] = jnp.broadcast_to(
        jnp.sum(pt, axis=0, keepdims=True), degb_ref.shape)


def _rowsum_body(pt_ref, o_ref):
    c = pl.program_id(0)

    @pl.when(c == 0)
    def _():
        o_ref[...] = jnp.zeros_like(o_ref)

    part = jnp.sum(pt_ref[...].astype(jnp.float32), axis=1, keepdims=True)
    o_ref[...] = o_ref[...] + jnp.broadcast_to(part, o_ref.shape)


def _scaled_mm_body(pt_ref, v_ref, sc_ref, o_ref, acc_ref, *, ksteps, dimnum):
    k = pl.program_id(1)

    @pl.when(k == 0)
    def _():
        acc_ref[...] = jnp.zeros_like(acc_ref)

    acc_ref[...] = acc_ref[...] + jax.lax.dot_general(
        pt_ref[...], v_ref[...], dimnum,
        preferred_element_type=jnp.float32)

    @pl.when(k == ksteps - 1)
    def _():
        o_ref[...] = (acc_ref[...] * sc_ref[...][:, :1]).astype(o_ref.dtype)


def _scaled_mm_tanh_body(pt_ref, v_ref, sc_ref, b_ref, o_ref, acc_ref, *,
                         ksteps, dimnum):
    k = pl.program_id(1)

    @pl.when(k == 0)
    def _():
        acc_ref[...] = jnp.zeros_like(acc_ref)

    acc_ref[...] = acc_ref[...] + jax.lax.dot_general(
        pt_ref[...], v_ref[...], dimnum,
        preferred_element_type=jnp.float32)

    @pl.when(k == ksteps - 1)
    def _():
        o_ref[...] = jnp.tanh(
            acc_ref[...] * sc_ref[...][:, :1] + b_ref[0:1, :])


def _dense_relu_body(a_ref, w_ref, b_ref, o_ref):
    acc = jax.lax.dot_general(
        a_ref[...], w_ref[...], _DIMNUM_N,
        preferred_element_type=jnp.float32)
    o_ref[...] = jnp.maximum(acc + b_ref[0:1, :], 0.0).astype(o_ref.dtype)


def _mm_acc_body(a_ref, w_ref, o_ref, acc_ref, *, ksteps):
    k = pl.program_id(1)

    @pl.when(k == 0)
    def _():
        acc_ref[...] = jnp.zeros_like(acc_ref)

    acc_ref[...] = acc_ref[...] + jax.lax.dot_general(
        a_ref[...], w_ref[...], _DIMNUM_N,
        preferred_element_type=jnp.float32)

    @pl.when(k == ksteps - 1)
    def _():
        o_ref[...] = acc_ref[...].astype(o_ref.dtype)


def kernel(x, S, W1, b1, W2, b2):
    n = S.shape[0]
    f = x.shape[1]
    hid = W1.shape[1]
    code = W2.shape[1]
    cpad = 128

    bc = 128
    pt, degb8 = pl.pallas_call(
        _topk_body,
        grid=(n // bc,),
        in_specs=[pl.BlockSpec((n, bc), lambda c: (0, c))],
        out_specs=[pl.BlockSpec((n, bc), lambda c: (0, c)),
                   pl.BlockSpec((8, bc), lambda c: (0, c))],
        out_shape=[jax.ShapeDtypeStruct((n, n), jnp.bfloat16),
                   jax.ShapeDtypeStruct((8, n), jnp.float32)],
    )(S)

    bd = 256
    degd_w = pl.pallas_call(
        _rowsum_body,
        grid=(n // bd,),
        in_specs=[pl.BlockSpec((n, bd), lambda c: (0, c))],
        out_specs=pl.BlockSpec((n, 128), lambda c: (0, 0)),
        out_shape=jax.ShapeDtypeStruct((n, 128), jnp.float32),
    )(pt)

    degb = degb8[0]
    degd = degd_w[:, 0]
    binv = jnp.where(degb > 0, 1.0 / degb, 0.0)
    dinv = jnp.where(degd > 0, 1.0 / degd, 0.0)
    binv2 = jnp.broadcast_to(binv[:, None], (n, 128))
    dinv2 = jnp.broadcast_to(dinv[:, None], (n, 128))

    bi = 1024
    bk = 1024
    ks = n // bk
    x_b = x.astype(jnp.bfloat16)
    w1_b = W1.astype(jnp.bfloat16)

    he1 = pl.pallas_call(
        functools.partial(_scaled_mm_body, ksteps=ks, dimnum=_DIMNUM_T),
        grid=(n // bi, ks),
        in_specs=[pl.BlockSpec((bk, bi), lambda j, k: (k, j)),
                  pl.BlockSpec((bk, f), lambda j, k: (k, 0)),
                  pl.BlockSpec((bi, 128), lambda j, k: (j, 0))],
        out_specs=pl.BlockSpec((bi, f), lambda j, k: (j, 0)),
        out_shape=jax.ShapeDtypeStruct((n, f), jnp.bfloat16),
        scratch_shapes=[pltpu.VMEM((bi, f), jnp.float32)],
    )(pt, x_b, binv2)
    agg1 = pl.pallas_call(
        functools.partial(_scaled_mm_body, ksteps=ks, dimnum=_DIMNUM_N),
        grid=(n // bi, ks),
        in_specs=[pl.BlockSpec((bi, bk), lambda i, k: (i, k)),
                  pl.BlockSpec((bk, f), lambda i, k: (k, 0)),
                  pl.BlockSpec((bi, 128), lambda i, k: (i, 0))],
        out_specs=pl.BlockSpec((bi, f), lambda i, k: (i, 0)),
        out_shape=jax.ShapeDtypeStruct((n, f), jnp.bfloat16),
        scratch_shapes=[pltpu.VMEM((bi, f), jnp.float32)],
    )(pt, he1, dinv2)

    b1_2d = jnp.broadcast_to(b1[None, :], (8, hid))
    bj = 1024
    feat = pl.pallas_call(
        _dense_relu_body,
        grid=(n // bi, hid // bj),
        in_specs=[pl.BlockSpec((bi, f), lambda i, j: (i, 0)),
                  pl.BlockSpec((f, bj), lambda i, j: (0, j)),
                  pl.BlockSpec((8, bj), lambda i, j: (0, j))],
        out_specs=pl.BlockSpec((bi, bj), lambda i, j: (i, j)),
        out_shape=jax.ShapeDtypeStruct((n, hid), jnp.bfloat16),
    )(agg1, w1_b, b1_2d)

    w2p = jnp.pad(W2, ((0, 0), (0, cpad - code))).astype(jnp.bfloat16)
    b2p = jnp.broadcast_to(jnp.pad(b2, (0, cpad - code))[None, :], (8, cpad))
    xt2 = pl.pallas_call(
        functools.partial(_mm_acc_body, ksteps=hid // bk),
        grid=(n // bi, hid // bk),
        in_specs=[pl.BlockSpec((bi, bk), lambda i, k: (i, k)),
                  pl.BlockSpec((bk, cpad), lambda i, k: (k, 0))],
        out_specs=pl.BlockSpec((bi, cpad), lambda i, k: (i, 0)),
        out_shape=jax.ShapeDtypeStruct((n, cpad), jnp.bfloat16),
        scratch_shapes=[pltpu.VMEM((bi, cpad), jnp.float32)],
    )(feat, w2p)
    he2 = pl.pallas_call(
        functools.partial(_scaled_mm_body, ksteps=ks, dimnum=_DIMNUM_T),
        grid=(n // bi, ks),
        in_specs=[pl.BlockSpec((bk, bi), lambda j, k: (k, j)),
                  pl.BlockSpec((bk, cpad), lambda j, k: (k, 0)),
                  pl.BlockSpec((bi, 128), lambda j, k: (j, 0))],
        out_specs=pl.BlockSpec((bi, cpad), lambda j, k: (j, 0)),
        out_shape=jax.ShapeDtypeStruct((n, cpad), jnp.bfloat16),
        scratch_shapes=[pltpu.VMEM((bi, cpad), jnp.float32)],
    )(pt, xt2, binv2)
    code_pad = pl.pallas_call(
        functools.partial(_scaled_mm_tanh_body, ksteps=ks, dimnum=_DIMNUM_N),
        grid=(n // bi, ks),
        in_specs=[pl.BlockSpec((bi, bk), lambda i, k: (i, k)),
                  pl.BlockSpec((bk, cpad), lambda i, k: (k, 0)),
                  pl.BlockSpec((bi, 128), lambda i, k: (i, 0)),
                  pl.BlockSpec((8, cpad), lambda i, k: (0, 0))],
        out_specs=pl.BlockSpec((bi, cpad), lambda i, k: (i, 0)),
        out_shape=jax.ShapeDtypeStruct((n, cpad), jnp.float32),
        scratch_shapes=[pltpu.VMEM((bi, cpad), jnp.float32)],
    )(pt, he2, dinv2, b2p)

    return code_pad[:, :code]
